# Initial kernel scaffold; baseline (speedup 1.0000x reference)
import jax
import jax.numpy as jnp
from jax.experimental import pallas as pl


def kernel(x_0, x_1, x_2, hodge_laplacian_0, hodge_laplacian_1, hodge_laplacian_2, y, batch_0, W0, W1, W2, ln0_g, ln0_b, ln1_g, ln1_b, ln2_g, ln2_b):
    raise NotImplementedError("write your pallas kernel here")


# speedup vs baseline: 1.1165x; 1.1165x over previous
import functools

import jax
import jax.numpy as jnp
from jax.experimental import pallas as pl
from jax.experimental.pallas import tpu as pltpu


def _scn_block(h_ref, x_ref, w_ref, g_ref, b_ref, o_ref, u_s, inv_s, *, R):
    p = pl.program_id(0)
    i = pl.program_id(1)

    @pl.when(p == 0)
    def _rowsum_phase():
        strip = h_ref[...]
        s = jnp.sum(jnp.abs(strip), axis=1, keepdims=True)
        inv = jnp.where(s > 0, jax.lax.rsqrt(s), 0.0)
        inv_s[pl.ds(i * R, R), :] = inv
        z = jnp.dot(x_ref[...], w_ref[...],
                    preferred_element_type=jnp.float32)
        u_s[pl.ds(i * R, R), :] = inv * z

    @pl.when(p == 1)
    def _matmul_phase():
        strip = h_ref[...]
        acc = jax.lax.dot_general(
            strip, u_s[...], (((1,), (0,)), ((), ())),
            preferred_element_type=jnp.float32)
        inv = inv_s[pl.ds(i * R, R), :]
        h = jax.nn.relu(acc * inv) + x_ref[...]
        mu = jnp.mean(h, axis=1, keepdims=True)
        var = jnp.mean((h - mu) ** 2, axis=1, keepdims=True)
        o_ref[...] = ((h - mu) * jax.lax.rsqrt(var + 1e-5)
                      * g_ref[...] + b_ref[...])


def _scn_layer(h, x, w, g, b, R):
    n, d = x.shape
    grid = (2, n // R)
    return pl.pallas_call(
        functools.partial(_scn_block, R=R),
        grid=grid,
        in_specs=[
            pl.BlockSpec((R, n), lambda p, i: (i, 0)),
            pl.BlockSpec((R, d), lambda p, i: (i, 0)),
            pl.BlockSpec((d, d), lambda p, i: (0, 0)),
            pl.BlockSpec((1, d), lambda p, i: (0, 0)),
            pl.BlockSpec((1, d), lambda p, i: (0, 0)),
        ],
        out_specs=pl.BlockSpec((R, d), lambda p, i: (i * p, 0)),
        out_shape=jax.ShapeDtypeStruct((n, d), jnp.float32),
        scratch_shapes=[
            pltpu.VMEM((n, d), jnp.float32),
            pltpu.VMEM((n, 1), jnp.float32),
        ],
    )(h, x, w, g.reshape(1, d), b.reshape(1, d))


def kernel(x_0, x_1, x_2, hodge_laplacian_0, hodge_laplacian_1,
           hodge_laplacian_2, y, batch_0, W0, W1, W2,
           ln0_g, ln0_b, ln1_g, ln1_b, ln2_g, ln2_b):
    out0 = _scn_layer(hodge_laplacian_0, x_0, W0, ln0_g, ln0_b, R=512)
    out1 = _scn_layer(hodge_laplacian_1, x_1, W1, ln1_g, ln1_b, R=512)
    out2 = _scn_layer(hodge_laplacian_2, x_2, W2, ln2_g, ln2_b, R=512)
    return (out0, out1, out2)


# speedup vs baseline: 1.1200x; 1.0031x over previous
import functools

import jax
import jax.numpy as jnp
from jax.experimental import pallas as pl
from jax.experimental.pallas import tpu as pltpu


def _scn_block(h_ref, x_ref, w_ref, g_ref, b_ref, o_ref, u_s, inv_s, *, R):
    p = pl.program_id(0)
    i = pl.program_id(1)

    @pl.when(p == 0)
    def _rowsum_phase():
        strip = h_ref[...]
        s = jnp.sum(jnp.abs(strip), axis=1, keepdims=True)
        inv = jnp.where(s > 0, jax.lax.rsqrt(s), 0.0)
        inv_s[pl.ds(i * R, R), :] = inv
        z = jnp.dot(x_ref[...], w_ref[...],
                    preferred_element_type=jnp.float32)
        u_s[pl.ds(i * R, R), :] = (inv * z).astype(jnp.bfloat16)

    @pl.when(p == 1)
    def _matmul_phase():
        strip = h_ref[...].astype(jnp.bfloat16)
        acc = jax.lax.dot_general(
            strip, u_s[...], (((1,), (0,)), ((), ())),
            preferred_element_type=jnp.float32)
        inv = inv_s[pl.ds(i * R, R), :]
        h = jax.nn.relu(acc * inv) + x_ref[...]
        mu = jnp.mean(h, axis=1, keepdims=True)
        var = jnp.mean((h - mu) ** 2, axis=1, keepdims=True)
        o_ref[...] = ((h - mu) * jax.lax.rsqrt(var + 1e-5)
                      * g_ref[...] + b_ref[...])


def _scn_layer(h, x, w, g, b, R):
    n, d = x.shape
    grid = (2, n // R)
    return pl.pallas_call(
        functools.partial(_scn_block, R=R),
        grid=grid,
        in_specs=[
            pl.BlockSpec((R, n), lambda p, i: (i, 0)),
            pl.BlockSpec((R, d), lambda p, i: (i, 0)),
            pl.BlockSpec((d, d), lambda p, i: (0, 0)),
            pl.BlockSpec((1, d), lambda p, i: (0, 0)),
            pl.BlockSpec((1, d), lambda p, i: (0, 0)),
        ],
        out_specs=pl.BlockSpec((R, d), lambda p, i: (i * p, 0)),
        out_shape=jax.ShapeDtypeStruct((n, d), jnp.float32),
        scratch_shapes=[
            pltpu.VMEM((n, d), jnp.bfloat16),
            pltpu.VMEM((n, 1), jnp.float32),
        ],
    )(h, x, w, g.reshape(1, d), b.reshape(1, d))


def kernel(x_0, x_1, x_2, hodge_laplacian_0, hodge_laplacian_1,
           hodge_laplacian_2, y, batch_0, W0, W1, W2,
           ln0_g, ln0_b, ln1_g, ln1_b, ln2_g, ln2_b):
    out0 = _scn_layer(hodge_laplacian_0, x_0, W0, ln0_g, ln0_b, R=512)
    out1 = _scn_layer(hodge_laplacian_1, x_1, W1, ln1_g, ln1_b, R=512)
    out2 = _scn_layer(hodge_laplacian_2, x_2, W2, ln2_g, ln2_b, R=512)
    return (out0, out1, out2)
